# baseline (speedup 1.0000x reference)
import functools

import jax
import jax.numpy as jnp
from jax import lax
from jax.experimental import pallas as pl
from jax.experimental.pallas import tpu as pltpu

N_LAB = 10
T = 20


def _body(lab_sref, x_ref, rates_ref, assign_ref, acc_ref, s_ref, cnt_ref,
          *, grid, bb):
    i = pl.program_id(0)

    @pl.when(i == 0)
    def _init():
        acc_ref[...] = jnp.zeros_like(acc_ref)
        for l in range(N_LAB):
            cnt_ref[l] = 0

    groups = []
    for g in range(T // 4):
        gs = x_ref[:, 4 * g]
        for t in range(4 * g + 1, 4 * g + 4):
            gs = gs + x_ref[:, t]
        groups.append(gs)
    s = groups[0]
    for g in range(1, T // 4):
        s = s + groups[g]
    s_ref[...] = s

    def body(b, carry):
        lab = lab_sref[i * bb + b]
        acc_ref[lab] = acc_ref[lab] + s_ref[b]
        cnt_ref[lab] = cnt_ref[lab] + 1
        return carry

    lax.fori_loop(0, bb, body, 0)

    @pl.when(i == grid - 1)
    def _finish():
        means = []
        for l in range(N_LAB):
            c = cnt_ref[l]
            m_l = acc_ref[l] / jnp.maximum(c.astype(jnp.float32), 1.0)
            m_l = jnp.where(c > 0, m_l, 0.0)
            means.append(m_l)
            rates_ref[l] = m_l
        m = means[0]
        am = jnp.zeros(m.shape, dtype=jnp.int32)
        for l in range(1, N_LAB):
            gt = means[l] > m
            am = jnp.where(gt, l, am)
            m = jnp.where(gt, means[l], m)
        assign_ref[...] = am


@jax.jit
def kernel(spikes, labels):
    b, t, n = spikes.shape
    x4 = spikes.reshape(b, t, n // 128, 128)

    grid = 32
    bb = b // grid

    grid_spec = pltpu.PrefetchScalarGridSpec(
        num_scalar_prefetch=1,
        grid=(grid,),
        in_specs=[
            pl.BlockSpec((bb, t, n // 128, 128), lambda i, lab: (i, 0, 0, 0)),
        ],
        out_specs=[
            pl.BlockSpec((N_LAB, n // 128, 128), lambda i, lab: (0, 0, 0)),
            pl.BlockSpec((n // 128, 128), lambda i, lab: (0, 0)),
        ],
        scratch_shapes=[
            pltpu.VMEM((N_LAB, n // 128, 128), jnp.float32),
            pltpu.VMEM((bb, n // 128, 128), jnp.float32),
            pltpu.SMEM((N_LAB,), jnp.int32),
        ],
    )

    rates3, assign2 = pl.pallas_call(
        functools.partial(_body, grid=grid, bb=bb),
        grid_spec=grid_spec,
        out_shape=[
            jax.ShapeDtypeStruct((N_LAB, n // 128, 128), jnp.float32),
            jax.ShapeDtypeStruct((n // 128, 128), jnp.int32),
        ],
    )(labels, x4)

    rates = rates3.reshape(N_LAB, n).T
    assignments = assign2.reshape(n)
    return assignments, rates
